# baseline (speedup 1.0000x reference)
import functools

import jax
import jax.numpy as jnp
from jax import lax
from jax.experimental import pallas as pl
from jax.experimental.pallas import tpu as pltpu
from jax.experimental.pallas import tpu_sc as plsc

D = 64
NB = 4096
NH = 200
B_TOTAL = NB * NH
NC, NS = 2, 16
NW = NC * NS
B_PER_W = B_TOTAL // NW
GCHUNK = 400
N_CHUNKS = B_PER_W // GCHUNK
NBUF = 4

_mesh = plsc.VectorSubcoreMesh(core_axis_name="c", subcore_axis_name="s")


@functools.partial(
    pl.kernel,
    mesh=_mesh,
    out_type=jax.ShapeDtypeStruct((B_TOTAL, D), jnp.float32),
    scratch_types=[
        pltpu.VMEM((B_PER_W,), jnp.int32),
        pltpu.VMEM((GCHUNK, D), jnp.float32),
        pltpu.VMEM((GCHUNK, D), jnp.float32),
        pltpu.VMEM((GCHUNK, D), jnp.float32),
        pltpu.VMEM((GCHUNK, D), jnp.float32),
        pltpu.SemaphoreType.DMA,
        pltpu.SemaphoreType.DMA,
        pltpu.SemaphoreType.DMA,
        pltpu.SemaphoreType.DMA,
        pltpu.SemaphoreType.DMA,
        pltpu.SemaphoreType.DMA,
        pltpu.SemaphoreType.DMA,
        pltpu.SemaphoreType.DMA,
    ],
    compiler_params=pltpu.CompilerParams(use_tc_tiling_on_sc=False),
)
def _emb_gather(table_hbm, prof_hbm, out_hbm, idx_v,
                r0, r1, r2, r3,
                g0, g1, g2, g3, w0, w1, w2, w3):
    wid = lax.axis_index("s") * NC + lax.axis_index("c")
    base = wid * B_PER_W
    rows = (r0, r1, r2, r3)
    gsem = (g0, g1, g2, g3)
    wsem = (w0, w1, w2, w3)

    pltpu.sync_copy(prof_hbm.at[pl.ds(base, B_PER_W)], idx_v)

    def gather(c, b):
        pltpu.async_copy(
            table_hbm.at[idx_v.at[pl.ds(c * GCHUNK, GCHUNK)]],
            rows[b], gsem[b],
        )

    def gather_wait(c, b):
        pltpu.make_async_copy(
            table_hbm.at[idx_v.at[pl.ds(c * GCHUNK, GCHUNK)]],
            rows[b], gsem[b],
        ).wait()

    def write(c, b):
        pltpu.async_copy(rows[b], out_hbm.at[pl.ds(base + c * GCHUNK, GCHUNK)],
                         wsem[b])

    def write_wait(c, b):
        pltpu.make_async_copy(
            rows[b], out_hbm.at[pl.ds(base + c * GCHUNK, GCHUNK)], wsem[b],
        ).wait()

    gather(0, 0)
    gather(1, 1)
    gather(2, 2)
    gather_wait(0, 0)
    write(0, 0)
    gather(3, 3)

    def body(k, carry):
        for j in range(NBUF):
            c = NBUF * k + 1 + j
            b = (1 + j) % NBUF
            gather_wait(c, b)
            write(c, b)
            write_wait(c - 1, j)
            gather(c + 3, j)
        return carry

    lax.fori_loop(0, (N_CHUNKS - NBUF) // NBUF, body, 0)

    for c in range(N_CHUNKS - 3, N_CHUNKS):
        gather_wait(c, c % NBUF)
        write(c, c % NBUF)
    for c in range(N_CHUNKS - NBUF, N_CHUNKS):
        write_wait(c, c % NBUF)


def kernel(profile, table):
    b, h = profile.shape
    flat = profile.reshape(b * h)
    out = _emb_gather(table, flat)
    return out.reshape(b, h, D)


# speedup vs baseline: 1.0289x; 1.0289x over previous
import functools

import jax
import jax.numpy as jnp
from jax import lax
from jax.experimental import pallas as pl
from jax.experimental.pallas import tpu as pltpu
from jax.experimental.pallas import tpu_sc as plsc

D = 64
NB = 4096
NH = 200
B_TOTAL = NB * NH
NC, NS = 2, 16
NW = NC * NS
BSTRIP = NB // NW
N_CHUNKS = NH
GCHUNK = BSTRIP
NBUF = 4

_mesh = plsc.VectorSubcoreMesh(core_axis_name="c", subcore_axis_name="s")


@functools.partial(
    pl.kernel,
    mesh=_mesh,
    out_type=jax.ShapeDtypeStruct((B_TOTAL, D), jnp.float32),
    scratch_types=[
        pltpu.VMEM((NH, BSTRIP), jnp.int32),
        pltpu.VMEM((GCHUNK, D), jnp.float32),
        pltpu.VMEM((GCHUNK, D), jnp.float32),
        pltpu.VMEM((GCHUNK, D), jnp.float32),
        pltpu.VMEM((GCHUNK, D), jnp.float32),
        pltpu.SemaphoreType.DMA,
        pltpu.SemaphoreType.DMA,
        pltpu.SemaphoreType.DMA,
        pltpu.SemaphoreType.DMA,
        pltpu.SemaphoreType.DMA,
        pltpu.SemaphoreType.DMA,
        pltpu.SemaphoreType.DMA,
        pltpu.SemaphoreType.DMA,
    ],
    compiler_params=pltpu.CompilerParams(use_tc_tiling_on_sc=False),
)
def _emb_gather(table_hbm, prof_hbm, out_hbm, idx_v,
                r0, r1, r2, r3,
                g0, g1, g2, g3, w0, w1, w2, w3):
    wid = lax.axis_index("s") * NC + lax.axis_index("c")
    b0 = wid * BSTRIP
    rows = (r0, r1, r2, r3)
    gsem = (g0, g1, g2, g3)
    wsem = (w0, w1, w2, w3)

    pltpu.sync_copy(prof_hbm.at[pl.ds(0, NH), pl.ds(b0, BSTRIP)], idx_v)

    def gather(c, b):
        pltpu.async_copy(table_hbm.at[idx_v.at[c]], rows[b], gsem[b])

    def gather_wait(c, b):
        pltpu.make_async_copy(
            table_hbm.at[idx_v.at[c]], rows[b], gsem[b]).wait()

    def write(c, b):
        pltpu.async_copy(rows[b], out_hbm.at[pl.ds(c * NB + b0, GCHUNK)],
                         wsem[b])

    def write_wait(c, b):
        pltpu.make_async_copy(
            rows[b], out_hbm.at[pl.ds(c * NB + b0, GCHUNK)], wsem[b],
        ).wait()

    gather(0, 0)
    gather(1, 1)
    gather(2, 2)
    gather_wait(0, 0)
    write(0, 0)
    gather(3, 3)

    def body(k, carry):
        for j in range(NBUF):
            c = NBUF * k + 1 + j
            b = (1 + j) % NBUF
            gather_wait(c, b)
            write(c, b)
            write_wait(c - 1, j)
            gather(c + 3, j)
        return carry

    lax.fori_loop(0, (N_CHUNKS - NBUF) // NBUF, body, 0)

    for c in range(N_CHUNKS - 3, N_CHUNKS):
        gather_wait(c, c % NBUF)
        write(c, c % NBUF)
    for c in range(N_CHUNKS - NBUF, N_CHUNKS):
        write_wait(c, c % NBUF)


def kernel(profile, table):
    b, h = profile.shape
    out = _emb_gather(table, profile.T)
    return out.reshape(h, b, D).transpose(1, 0, 2)


# speedup vs baseline: 1.3721x; 1.3335x over previous
import functools

import jax
import jax.numpy as jnp
from jax import lax
from jax.experimental import pallas as pl
from jax.experimental.pallas import tpu as pltpu
from jax.experimental.pallas import tpu_sc as plsc

D = 64
NB = 4096
NH = 200
B_TOTAL = NB * NH
NC, NS = 2, 16
NW = NC * NS
BSTRIP = NB // NW
N_CHUNKS = NH
GCHUNK = BSTRIP
NBUF = 4

_mesh = plsc.VectorSubcoreMesh(core_axis_name="c", subcore_axis_name="s")


@functools.partial(
    pl.kernel,
    mesh=_mesh,
    out_type=jax.ShapeDtypeStruct((B_TOTAL, 2 * D), jnp.float32),
    scratch_types=[
        pltpu.VMEM((NH, BSTRIP), jnp.int32),
        pltpu.VMEM((GCHUNK, D), jnp.float32),
        pltpu.VMEM((GCHUNK, D), jnp.float32),
        pltpu.VMEM((GCHUNK, D), jnp.float32),
        pltpu.VMEM((GCHUNK, D), jnp.float32),
        pltpu.SemaphoreType.DMA,
        pltpu.SemaphoreType.DMA,
        pltpu.SemaphoreType.DMA,
        pltpu.SemaphoreType.DMA,
        pltpu.SemaphoreType.DMA,
        pltpu.SemaphoreType.DMA,
        pltpu.SemaphoreType.DMA,
        pltpu.SemaphoreType.DMA,
    ],
    compiler_params=pltpu.CompilerParams(use_tc_tiling_on_sc=False),
)
def _emb_gather(table_hbm, prof_hbm, out_hbm, idx_v,
                r0, r1, r2, r3,
                g0, g1, g2, g3, w0, w1, w2, w3):
    wid = lax.axis_index("s") * NC + lax.axis_index("c")
    b0 = wid * BSTRIP
    rows = (r0, r1, r2, r3)
    gsem = (g0, g1, g2, g3)
    wsem = (w0, w1, w2, w3)

    pltpu.sync_copy(prof_hbm.at[pl.ds(0, NH), pl.ds(b0, BSTRIP)], idx_v)

    def gather(c, b):
        pltpu.async_copy(table_hbm.at[idx_v.at[c]], rows[b], gsem[b])

    def gather_wait(c, b):
        pltpu.make_async_copy(
            table_hbm.at[idx_v.at[c]], rows[b], gsem[b]).wait()

    def write(c, b):
        pltpu.async_copy(
            rows[b],
            out_hbm.at[pl.ds(c * NB + b0, GCHUNK), pl.ds(0, D)], wsem[b])

    def write_wait(c, b):
        pltpu.make_async_copy(
            rows[b],
            out_hbm.at[pl.ds(c * NB + b0, GCHUNK), pl.ds(0, D)], wsem[b],
        ).wait()

    gather(0, 0)
    gather(1, 1)
    gather(2, 2)
    gather_wait(0, 0)
    write(0, 0)
    gather(3, 3)

    def body(k, carry):
        for j in range(NBUF):
            c = NBUF * k + 1 + j
            b = (1 + j) % NBUF
            gather_wait(c, b)
            write(c, b)
            write_wait(c - 1, j)
            gather(c + 3, j)
        return carry

    lax.fori_loop(0, (N_CHUNKS - NBUF) // NBUF, body, 0)

    for c in range(N_CHUNKS - 3, N_CHUNKS):
        gather_wait(c, c % NBUF)
        write(c, c % NBUF)
    for c in range(N_CHUNKS - NBUF, N_CHUNKS):
        write_wait(c, c % NBUF)


def kernel(profile, table):
    b, h = profile.shape
    out = _emb_gather(table, profile.T)
    return out.reshape(h, b, 2 * D)[:, :, :D].transpose(1, 0, 2)
